# baseline (speedup 1.0000x reference)
import jax
import jax.numpy as jnp
from jax import lax
from jax.experimental import pallas as pl
from jax.experimental.pallas import tpu as pltpu
from jax.experimental.pallas import tpu_sc as plsc

U_N_ = 60000
I_N_ = 40000
N_ = U_N_ + I_N_
D_ = 32
DH_ = 16
L_LAYERS_ = 3
E_ = 1600000
B_ = 128
NTILE_ = 16
G_ = 4

NB_TILE_ = 792
NB_ = NB_TILE_ * NTILE_
EPAD_ = NB_ * B_
NPAD_ = 100096
OROWS_ = NPAD_ // NTILE_
CB_ = 24
NCH_ = NB_TILE_ // CB_
MFULL_ = OROWS_ // B_
MREM_ = OROWS_ - MFULL_ * B_
MLAST_ = N_ - (NTILE_ - 1) * OROWS_ - MFULL_ * B_


def _body(emb0_t, src2, dst2, w2, wu, wi,
          tbl, out2, emb0_o,
          acc, src_st, dst_st, w_st,
          ra0, ra1, ra2, ra3, rb0, rb1, rb2, rb3, cw,
          gsa, gsb, ssa, ssb):
    c = lax.axis_index("c")
    s = lax.axis_index("s")
    half_off = c * NPAD_

    grp_a = [ra0, ra1, ra2, ra3]
    grp_b = [rb0, rb1, rb2, rb3]
    r0 = s * OROWS_

    def zero_buf(buf):
        zv = jnp.zeros((16,), jnp.float32)

        def zrow(i, _):
            buf[i] = zv
            return 0

        lax.fori_loop(0, B_, zrow, 0)

    def zero_acc_slice():
        zero_buf(rb3)

        def zchunk(ch, _):
            pltpu.sync_copy(rb3, acc.at[pl.ds(r0 + ch * B_, B_)])
            return 0

        lax.fori_loop(0, MFULL_, zchunk, 0)
        pltpu.sync_copy(rb3.at[pl.ds(0, MREM_)],
                        acc.at[pl.ds(r0 + MFULL_ * B_, MREM_)])

    zero_acc_slice()

    UR_, IR_ = U_N_ // NTILE_, I_N_ // NTILE_
    CWR_ = 125

    def emb0_user(ch, _):
        rr = s * UR_ + ch * CWR_
        pltpu.sync_copy(wu.at[pl.ds(rr, CWR_)], cw)
        pltpu.sync_copy(cw, emb0_o.at[pl.ds(rr, CWR_)])
        return 0

    def emb0_item(ch, _):
        rr = s * IR_ + ch * CWR_
        pltpu.sync_copy(wi.at[pl.ds(rr, CWR_)], cw)
        pltpu.sync_copy(cw, emb0_o.at[pl.ds(U_N_ + rr, CWR_)])
        return 0

    lax.fori_loop(0, UR_ // CWR_, emb0_user, 0)
    lax.fori_loop(0, IR_ // CWR_, emb0_item, 0)
    plsc.subcore_barrier()

    def scale(buf, jj):
        for m in range(B_ // 16):
            w16 = w_st[jj, pl.ds(m * 16, 16)]
            for k in range(16):
                i = m * 16 + k
                buf[i] = buf[i] * w16[k]

    def edge_chunks(src_tab):
        def g_start(jg, bufs, sem):
            for b in range(G_):
                pltpu.async_copy(src_tab.at[src_st.at[jg + b]], bufs[b], sem)

        def g_wait(bufs, sem):
            for b in range(G_):
                pltpu.make_async_copy(src_tab.at[src_st.at[0]], bufs[b],
                                      sem).wait()

        def s_start(jg, bufs, sem):
            for b in range(G_):
                pltpu.async_copy(bufs[b], acc.at[dst_st.at[jg + b]], sem,
                                 add=True)

        def s_wait(bufs, sem):
            for b in range(G_):
                pltpu.make_async_copy(bufs[b], acc.at[dst_st.at[0]],
                                      sem).wait()

        def scale4(jg, bufs):
            for b in range(G_):
                scale(bufs[b], jg + b)

        def chunk(cb, _):
            base = s * NB_TILE_ + cb * CB_
            pltpu.sync_copy(src2.at[pl.ds(base, CB_)], src_st)
            pltpu.sync_copy(dst2.at[pl.ds(base, CB_)], dst_st)
            pltpu.sync_copy(w2.at[pl.ds(base, CB_)], w_st)

            def add_off(jj, _):
                for k in range(B_ // 16):
                    sl = pl.ds(k * 16, 16)
                    src_st[jj, sl] = src_st[jj, sl] + half_off
                return 0

            lax.fori_loop(0, CB_, add_off, 0)

            nsp = CB_ // (2 * G_)
            g_start(0, grp_a, gsa)

            def superpair(t, _):
                jA = 2 * G_ * t
                jB = jA + G_
                g_wait(grp_a, gsa)

                @pl.when(t > 0)
                def _():
                    s_wait(grp_b, ssb)

                g_start(jB, grp_b, gsb)
                scale4(jA, grp_a)
                s_start(jA, grp_a, ssa)
                g_wait(grp_b, gsb)
                s_wait(grp_a, ssa)

                @pl.when(t < nsp - 1)
                def _():
                    g_start(jA + 2 * G_, grp_a, gsa)

                scale4(jB, grp_b)
                s_start(jB, grp_b, ssb)
                return 0

            lax.fori_loop(0, nsp, superpair, 0)
            s_wait(grp_b, ssb)
            return 0

        lax.fori_loop(0, NCH_, chunk, 0)

    def publish_and_rezero(layer):
        plsc.subcore_barrier()
        pltpu.sync_copy(acc.at[pl.ds(r0, OROWS_)],
                        tbl.at[layer, pl.ds(half_off + r0, OROWS_)])
        zero_acc_slice()
        plsc.subcore_barrier()

    edge_chunks(emb0_t)
    publish_and_rezero(0)

    def layer_step(lay, _):
        edge_chunks(tbl.at[lay])
        publish_and_rezero(lay + 1)
        return 0

    lax.fori_loop(0, L_LAYERS_ - 1, layer_step, 0)

    def mean_rows(nrows):
        def body(i, _):
            rb0[i] = (ra0[i] + ra1[i] + ra2[i] + ra3[i]) * 0.25
            return 0

        lax.fori_loop(0, nrows, body, 0)

    col0 = c * DH_

    def mean_chunk(ch, _):
        g0 = half_off + r0 + ch * B_
        gr = r0 + ch * B_
        pltpu.sync_copy(emb0_t.at[pl.ds(g0, B_)], ra0)
        pltpu.sync_copy(tbl.at[0, pl.ds(g0, B_)], ra1)
        pltpu.sync_copy(tbl.at[1, pl.ds(g0, B_)], ra2)
        pltpu.sync_copy(tbl.at[2, pl.ds(g0, B_)], ra3)
        mean_rows(B_)
        pltpu.sync_copy(rb0, out2.at[pl.ds(gr, B_), pl.ds(col0, DH_)])
        return 0

    lax.fori_loop(0, MFULL_, mean_chunk, 0)
    g0 = half_off + r0 + MFULL_ * B_
    gr = r0 + MFULL_ * B_
    pltpu.sync_copy(emb0_t.at[pl.ds(g0, MREM_)], ra0.at[pl.ds(0, MREM_)])
    pltpu.sync_copy(tbl.at[0, pl.ds(g0, MREM_)], ra1.at[pl.ds(0, MREM_)])
    pltpu.sync_copy(tbl.at[1, pl.ds(g0, MREM_)], ra2.at[pl.ds(0, MREM_)])
    pltpu.sync_copy(tbl.at[2, pl.ds(g0, MREM_)], ra3.at[pl.ds(0, MREM_)])
    mean_rows(MREM_)

    @pl.when(s < NTILE_ - 1)
    def _():
        pltpu.sync_copy(rb0.at[pl.ds(0, MREM_)],
                        out2.at[pl.ds(gr, MREM_), pl.ds(col0, DH_)])

    @pl.when(s == NTILE_ - 1)
    def _():
        pltpu.sync_copy(rb0.at[pl.ds(0, MLAST_)],
                        out2.at[pl.ds(gr, MLAST_), pl.ds(col0, DH_)])


@jax.jit
def _run(emb0_t, src2, dst2, w2, wu, wi):
    mesh = plsc.VectorSubcoreMesh(core_axis_name="c", subcore_axis_name="s")
    f32 = jnp.float32
    out_types = (
        jax.ShapeDtypeStruct((3, 2 * NPAD_, DH_), f32),
        jax.ShapeDtypeStruct((N_, D_), f32),
        jax.ShapeDtypeStruct((N_, D_), f32),
    )
    scratch = [
        pltpu.VMEM_SHARED((NPAD_, DH_), f32),
        pltpu.VMEM((CB_, B_), jnp.int32),
        pltpu.VMEM((CB_, B_), jnp.int32),
        pltpu.VMEM((CB_, B_), f32),
    ]
    scratch += [pltpu.VMEM((B_, DH_), f32) for _ in range(8)]
    scratch += [pltpu.VMEM((125, D_), f32)]
    scratch += [pltpu.SemaphoreType.DMA] * 4
    kfn = pl.kernel(
        _body,
        out_type=out_types,
        scratch_types=scratch,
        mesh=mesh,
        compiler_params=pltpu.CompilerParams(use_tc_tiling_on_sc=False),
    )
    return kfn(emb0_t, src2, dst2, w2, wu, wi)


def kernel(edge_index, edge_weight, W_user, W_item):
    rpad = jnp.zeros((NPAD_ - N_, DH_), jnp.float32)
    emb0_t = jnp.concatenate(
        [W_user[:, :DH_], W_item[:, :DH_], rpad,
         W_user[:, DH_:], W_item[:, DH_:], rpad], axis=0)

    src = edge_index[1]
    dst = edge_index[0]
    w = edge_weight
    pad = EPAD_ - E_
    src_p = jnp.concatenate([src, jnp.zeros((pad,), jnp.int32)])
    dst_p = jnp.concatenate([dst, jnp.full((pad,), N_, jnp.int32)])
    w_p = jnp.concatenate([w, jnp.zeros((pad,), jnp.float32)])
    src2 = src_p.reshape(NB_, B_)
    dst2 = dst_p.reshape(NB_, B_)
    w2 = w_p.reshape(NB_, B_)

    _, out, emb0 = _run(emb0_t, src2, dst2, w2, W_user, W_item)
    return (emb0, out)


# speedup vs baseline: 1.0584x; 1.0584x over previous
import jax
import jax.numpy as jnp
from jax import lax
from jax.experimental import pallas as pl
from jax.experimental.pallas import tpu as pltpu
from jax.experimental.pallas import tpu_sc as plsc

U_N_ = 60000
I_N_ = 40000
N_ = U_N_ + I_N_
D_ = 32
DH_ = 16
L_LAYERS_ = 3
E_ = 1600000
B_ = 128
NTILE_ = 16
G_ = 4

NB_TILE_ = 792
NB_ = NB_TILE_ * NTILE_
EPAD_ = NB_ * B_
NPAD_ = 100096
OROWS_ = NPAD_ // NTILE_
CB_ = 24
NCH_ = NB_TILE_ // CB_
MFULL_ = OROWS_ // B_
MREM_ = OROWS_ - MFULL_ * B_
MLAST_ = N_ - (NTILE_ - 1) * OROWS_ - MFULL_ * B_


def _body(emb0_t, src2, dst2, w2,
          tbl, out2,
          acc, src_st, dst_st, w_st,
          ra0, ra1, ra2, ra3, rb0, rb1, rb2, rb3,
          gsa, gsb, ssa, ssb):
    c = lax.axis_index("c")
    s = lax.axis_index("s")
    half_off = c * NPAD_

    grp_a = [ra0, ra1, ra2, ra3]
    grp_b = [rb0, rb1, rb2, rb3]
    r0 = s * OROWS_

    def zero_buf(buf):
        zv = jnp.zeros((16,), jnp.float32)

        def zrow(i, _):
            buf[i] = zv
            return 0

        lax.fori_loop(0, B_, zrow, 0)

    def zero_acc_slice():
        zero_buf(rb3)

        def zchunk(ch, _):
            pltpu.sync_copy(rb3, acc.at[pl.ds(r0 + ch * B_, B_)])
            return 0

        lax.fori_loop(0, MFULL_, zchunk, 0)
        pltpu.sync_copy(rb3.at[pl.ds(0, MREM_)],
                        acc.at[pl.ds(r0 + MFULL_ * B_, MREM_)])

    zero_acc_slice()

    plsc.subcore_barrier()

    def scale(buf, jj):
        for m in range(B_ // 16):
            w16 = w_st[jj, pl.ds(m * 16, 16)]
            for k in range(16):
                i = m * 16 + k
                buf[i] = buf[i] * w16[k]

    def edge_chunks(src_tab_full):
        src_tab = src_tab_full.at[pl.ds(pl.multiple_of(half_off, 8), NPAD_)]
        def g_start(jg, bufs, sem):
            for b in range(G_):
                pltpu.async_copy(src_tab.at[src_st.at[jg + b]], bufs[b], sem)

        def g_wait(bufs, sem):
            for b in range(G_):
                pltpu.make_async_copy(src_tab.at[src_st.at[0]], bufs[b],
                                      sem).wait()

        def s_start(jg, bufs, sem):
            for b in range(G_):
                pltpu.async_copy(bufs[b], acc.at[dst_st.at[jg + b]], sem,
                                 add=True)

        def s_wait(bufs, sem):
            for b in range(G_):
                pltpu.make_async_copy(bufs[b], acc.at[dst_st.at[0]],
                                      sem).wait()

        def scale4(jg, bufs):
            for b in range(G_):
                scale(bufs[b], jg + b)

        def chunk(cb, _):
            base = s * NB_TILE_ + cb * CB_
            pltpu.sync_copy(src2.at[pl.ds(base, CB_)], src_st)
            pltpu.sync_copy(dst2.at[pl.ds(base, CB_)], dst_st)
            pltpu.sync_copy(w2.at[pl.ds(base, CB_)], w_st)

            nsp = CB_ // (2 * G_)
            g_start(0, grp_a, gsa)

            def superpair(t, _):
                jA = 2 * G_ * t
                jB = jA + G_
                g_wait(grp_a, gsa)

                @pl.when(t > 0)
                def _():
                    s_wait(grp_b, ssb)

                g_start(jB, grp_b, gsb)
                scale4(jA, grp_a)
                s_start(jA, grp_a, ssa)
                g_wait(grp_b, gsb)
                s_wait(grp_a, ssa)

                @pl.when(t < nsp - 1)
                def _():
                    g_start(jA + 2 * G_, grp_a, gsa)

                scale4(jB, grp_b)
                s_start(jB, grp_b, ssb)
                return 0

            lax.fori_loop(0, nsp, superpair, 0)
            s_wait(grp_b, ssb)
            return 0

        lax.fori_loop(0, NCH_, chunk, 0)

    def publish_and_rezero(layer):
        plsc.subcore_barrier()
        pltpu.sync_copy(acc.at[pl.ds(r0, OROWS_)],
                        tbl.at[layer, pl.ds(half_off + r0, OROWS_)])
        zero_acc_slice()
        plsc.subcore_barrier()

    edge_chunks(emb0_t)
    publish_and_rezero(0)

    def layer_step(lay, _):
        edge_chunks(tbl.at[lay])
        publish_and_rezero(lay + 1)
        return 0

    lax.fori_loop(0, L_LAYERS_ - 1, layer_step, 0)

    def mean_rows(nrows):
        def body(i, _):
            rb0[i] = (ra0[i] + ra1[i] + ra2[i] + ra3[i]) * 0.25
            return 0

        lax.fori_loop(0, nrows, body, 0)

    col0 = c * DH_

    def mean_chunk(ch, _):
        g0 = half_off + r0 + ch * B_
        gr = r0 + ch * B_
        pltpu.sync_copy(emb0_t.at[pl.ds(g0, B_)], ra0)
        pltpu.sync_copy(tbl.at[0, pl.ds(g0, B_)], ra1)
        pltpu.sync_copy(tbl.at[1, pl.ds(g0, B_)], ra2)
        pltpu.sync_copy(tbl.at[2, pl.ds(g0, B_)], ra3)
        mean_rows(B_)
        pltpu.sync_copy(rb0, out2.at[pl.ds(gr, B_), pl.ds(col0, DH_)])
        return 0

    lax.fori_loop(0, MFULL_, mean_chunk, 0)
    g0 = half_off + r0 + MFULL_ * B_
    gr = r0 + MFULL_ * B_
    pltpu.sync_copy(emb0_t.at[pl.ds(g0, MREM_)], ra0.at[pl.ds(0, MREM_)])
    pltpu.sync_copy(tbl.at[0, pl.ds(g0, MREM_)], ra1.at[pl.ds(0, MREM_)])
    pltpu.sync_copy(tbl.at[1, pl.ds(g0, MREM_)], ra2.at[pl.ds(0, MREM_)])
    pltpu.sync_copy(tbl.at[2, pl.ds(g0, MREM_)], ra3.at[pl.ds(0, MREM_)])
    mean_rows(MREM_)

    @pl.when(s < NTILE_ - 1)
    def _():
        pltpu.sync_copy(rb0.at[pl.ds(0, MREM_)],
                        out2.at[pl.ds(gr, MREM_), pl.ds(col0, DH_)])

    @pl.when(s == NTILE_ - 1)
    def _():
        pltpu.sync_copy(rb0.at[pl.ds(0, MLAST_)],
                        out2.at[pl.ds(gr, MLAST_), pl.ds(col0, DH_)])


@jax.jit
def _run(emb0_t, src2, dst2, w2):
    mesh = plsc.VectorSubcoreMesh(core_axis_name="c", subcore_axis_name="s")
    f32 = jnp.float32
    out_types = (
        jax.ShapeDtypeStruct((3, 2 * NPAD_, DH_), f32),
        jax.ShapeDtypeStruct((N_, D_), f32),
    )
    scratch = [
        pltpu.VMEM_SHARED((NPAD_, DH_), f32),
        pltpu.VMEM((CB_, B_), jnp.int32),
        pltpu.VMEM((CB_, B_), jnp.int32),
        pltpu.VMEM((CB_, B_), f32),
    ]
    scratch += [pltpu.VMEM((B_, DH_), f32) for _ in range(8)]
    scratch += [pltpu.SemaphoreType.DMA] * 4
    kfn = pl.kernel(
        _body,
        out_type=out_types,
        scratch_types=scratch,
        mesh=mesh,
        compiler_params=pltpu.CompilerParams(use_tc_tiling_on_sc=False),
    )
    return kfn(emb0_t, src2, dst2, w2)


def kernel(edge_index, edge_weight, W_user, W_item):
    emb0 = jnp.concatenate([W_user, W_item], axis=0)
    rpad = jnp.zeros((NPAD_ - N_, DH_), jnp.float32)
    emb0_t = jnp.concatenate(
        [emb0[:, :DH_], rpad, emb0[:, DH_:], rpad], axis=0)

    src = edge_index[1]
    dst = edge_index[0]
    w = edge_weight
    pad = EPAD_ - E_
    src_p = jnp.concatenate([src, jnp.zeros((pad,), jnp.int32)])
    dst_p = jnp.concatenate([dst, jnp.full((pad,), N_, jnp.int32)])
    w_p = jnp.concatenate([w, jnp.zeros((pad,), jnp.float32)])
    src2 = src_p.reshape(NB_, B_)
    dst2 = dst_p.reshape(NB_, B_)
    w2 = w_p.reshape(NB_, B_)

    _, out = _run(emb0_t, src2, dst2, w2)
    return (emb0, out)


# speedup vs baseline: 1.2631x; 1.1935x over previous
import jax
import jax.numpy as jnp
from jax import lax
from jax.experimental import pallas as pl
from jax.experimental.pallas import tpu as pltpu
from jax.experimental.pallas import tpu_sc as plsc

U_N_ = 60000
I_N_ = 40000
N_ = U_N_ + I_N_
D_ = 32
DH_ = 16
L_LAYERS_ = 3
E_ = 1600000
B_ = 128
NTILE_ = 16
G_ = 4

NB_TILE_ = 792
NB_ = NB_TILE_ * NTILE_
EPAD_ = NB_ * B_
NPAD_ = 100096
OROWS_ = NPAD_ // NTILE_
CB2_ = 2 * G_
NCH2_ = NB_TILE_ // CB2_
MFULL_ = OROWS_ // B_
MREM_ = OROWS_ - MFULL_ * B_
MLAST_ = N_ - (NTILE_ - 1) * OROWS_ - MFULL_ * B_


def _body(emb0_t, src2, dst2, w2,
          tbl, out2,
          acc, src_st, dst_st, w_st,
          ra0, ra1, ra2, ra3, rb0, rb1, rb2, rb3,
          gsa, gsb, ssa, ssb, stg):
    c = lax.axis_index("c")
    s = lax.axis_index("s")
    half_off = c * NPAD_

    grp_a = [ra0, ra1, ra2, ra3]
    grp_b = [rb0, rb1, rb2, rb3]
    r0 = s * OROWS_

    def zero_buf(buf):
        zv = jnp.zeros((16,), jnp.float32)

        def zrow(i, _):
            buf[i] = zv
            return 0

        lax.fori_loop(0, B_, zrow, 0)

    def zero_acc_slice():
        zero_buf(rb3)

        def zchunk(ch, _):
            pltpu.sync_copy(rb3, acc.at[pl.ds(r0 + ch * B_, B_)])
            return 0

        lax.fori_loop(0, MFULL_, zchunk, 0)
        pltpu.sync_copy(rb3.at[pl.ds(0, MREM_)],
                        acc.at[pl.ds(r0 + MFULL_ * B_, MREM_)])

    zero_acc_slice()

    plsc.subcore_barrier()

    def scale(buf, slot, jj):
        for m in range(B_ // 16):
            w16 = w_st[slot, jj, pl.ds(m * 16, 16)]
            for k in range(16):
                i = m * 16 + k
                buf[i] = buf[i] * w16[k]

    def edge_chunks(src_tab_full):
        src_tab = src_tab_full.at[pl.ds(pl.multiple_of(half_off, 8), NPAD_)]

        tb = s * NB_TILE_

        def g_start(slot, jg, bufs, sem):
            for b in range(G_):
                pltpu.async_copy(src_tab.at[src_st.at[slot, jg + b]],
                                 bufs[b], sem)

        def g_wait(bufs, sem):
            for b in range(G_):
                pltpu.make_async_copy(src_tab.at[src_st.at[0, 0]], bufs[b],
                                      sem).wait()

        def s_start(slot, jg, bufs, sem):
            for b in range(G_):
                pltpu.async_copy(bufs[b], acc.at[dst_st.at[slot, jg + b]],
                                 sem, add=True)

        def s_wait(bufs, sem):
            for b in range(G_):
                pltpu.make_async_copy(bufs[b], acc.at[dst_st.at[0, 0]],
                                      sem).wait()

        def scale4(slot, jg, bufs):
            for b in range(G_):
                scale(bufs[b], slot, jg + b)

        def stage_sync(q, slot):
            pltpu.sync_copy(src2.at[pl.ds(tb + q * CB2_, CB2_)],
                            src_st.at[slot])
            pltpu.sync_copy(dst2.at[pl.ds(tb + q * CB2_, CB2_)],
                            dst_st.at[slot])
            pltpu.sync_copy(w2.at[pl.ds(tb + q * CB2_, CB2_)],
                            w_st.at[slot])

        def stage_async(q):
            slot = lax.rem(q, 3)
            pltpu.async_copy(src2.at[pl.ds(tb + q * CB2_, CB2_)],
                             src_st.at[slot], stg)
            pltpu.async_copy(dst2.at[pl.ds(tb + q * CB2_, CB2_)],
                             dst_st.at[slot], stg)
            pltpu.async_copy(w2.at[pl.ds(tb + q * CB2_, CB2_)],
                             w_st.at[slot], stg)

        def stage_wait():
            pltpu.make_async_copy(src2.at[pl.ds(tb, CB2_)], src_st.at[0],
                                  stg).wait()
            pltpu.make_async_copy(dst2.at[pl.ds(tb, CB2_)], dst_st.at[0],
                                  stg).wait()
            pltpu.make_async_copy(w2.at[pl.ds(tb, CB2_)], w_st.at[0],
                                  stg).wait()

        stage_sync(0, 0)
        stage_async(1)
        g_start(0, 0, grp_a, gsa)

        def superpair(t, _):
            slot = lax.rem(t, 3)
            g_wait(grp_a, gsa)

            @pl.when(t > 0)
            def _():
                s_wait(grp_b, ssb)

            @pl.when(t + 2 <= NCH2_ - 1)
            def _():
                stage_async(t + 2)

            g_start(slot, G_, grp_b, gsb)
            scale4(slot, 0, grp_a)
            s_start(slot, 0, grp_a, ssa)
            g_wait(grp_b, gsb)
            s_wait(grp_a, ssa)

            @pl.when(t < NCH2_ - 1)
            def _():
                stage_wait()
                g_start(lax.rem(t + 1, 3), 0, grp_a, gsa)

            scale4(slot, G_, grp_b)
            s_start(slot, G_, grp_b, ssb)
            return 0

        lax.fori_loop(0, NCH2_, superpair, 0)
        s_wait(grp_b, ssb)

    def publish_and_rezero(layer):
        plsc.subcore_barrier()
        pltpu.sync_copy(acc.at[pl.ds(r0, OROWS_)],
                        tbl.at[layer, pl.ds(half_off + r0, OROWS_)])
        zero_acc_slice()
        plsc.subcore_barrier()

    edge_chunks(emb0_t)
    publish_and_rezero(0)

    def layer_step(lay, _):
        edge_chunks(tbl.at[lay])
        publish_and_rezero(lay + 1)
        return 0

    lax.fori_loop(0, L_LAYERS_ - 1, layer_step, 0)

    def mean_rows(nrows):
        def body(i, _):
            rb0[i] = (ra0[i] + ra1[i] + ra2[i] + ra3[i]) * 0.25
            return 0

        lax.fori_loop(0, nrows, body, 0)

    col0 = c * DH_

    def mean_chunk(ch, _):
        g0 = half_off + r0 + ch * B_
        gr = r0 + ch * B_
        pltpu.sync_copy(emb0_t.at[pl.ds(g0, B_)], ra0)
        pltpu.sync_copy(tbl.at[0, pl.ds(g0, B_)], ra1)
        pltpu.sync_copy(tbl.at[1, pl.ds(g0, B_)], ra2)
        pltpu.sync_copy(tbl.at[2, pl.ds(g0, B_)], ra3)
        mean_rows(B_)
        pltpu.sync_copy(rb0, out2.at[pl.ds(gr, B_), pl.ds(col0, DH_)])
        return 0

    lax.fori_loop(0, MFULL_, mean_chunk, 0)
    g0 = half_off + r0 + MFULL_ * B_
    gr = r0 + MFULL_ * B_
    pltpu.sync_copy(emb0_t.at[pl.ds(g0, MREM_)], ra0.at[pl.ds(0, MREM_)])
    pltpu.sync_copy(tbl.at[0, pl.ds(g0, MREM_)], ra1.at[pl.ds(0, MREM_)])
    pltpu.sync_copy(tbl.at[1, pl.ds(g0, MREM_)], ra2.at[pl.ds(0, MREM_)])
    pltpu.sync_copy(tbl.at[2, pl.ds(g0, MREM_)], ra3.at[pl.ds(0, MREM_)])
    mean_rows(MREM_)

    @pl.when(s < NTILE_ - 1)
    def _():
        pltpu.sync_copy(rb0.at[pl.ds(0, MREM_)],
                        out2.at[pl.ds(gr, MREM_), pl.ds(col0, DH_)])

    @pl.when(s == NTILE_ - 1)
    def _():
        pltpu.sync_copy(rb0.at[pl.ds(0, MLAST_)],
                        out2.at[pl.ds(gr, MLAST_), pl.ds(col0, DH_)])


@jax.jit
def _run(emb0_t, src2, dst2, w2):
    mesh = plsc.VectorSubcoreMesh(core_axis_name="c", subcore_axis_name="s")
    f32 = jnp.float32
    out_types = (
        jax.ShapeDtypeStruct((3, 2 * NPAD_, DH_), f32),
        jax.ShapeDtypeStruct((N_, D_), f32),
    )
    scratch = [
        pltpu.VMEM_SHARED((NPAD_, DH_), f32),
        pltpu.VMEM((3, CB2_, B_), jnp.int32),
        pltpu.VMEM((3, CB2_, B_), jnp.int32),
        pltpu.VMEM((3, CB2_, B_), f32),
    ]
    scratch += [pltpu.VMEM((B_, DH_), f32) for _ in range(8)]
    scratch += [pltpu.SemaphoreType.DMA] * 5
    kfn = pl.kernel(
        _body,
        out_type=out_types,
        scratch_types=scratch,
        mesh=mesh,
        compiler_params=pltpu.CompilerParams(use_tc_tiling_on_sc=False),
    )
    return kfn(emb0_t, src2, dst2, w2)


def kernel(edge_index, edge_weight, W_user, W_item):
    emb0 = jnp.concatenate([W_user, W_item], axis=0)
    rpad = jnp.zeros((NPAD_ - N_, DH_), jnp.float32)
    emb0_t = jnp.concatenate(
        [emb0[:, :DH_], rpad, emb0[:, DH_:], rpad], axis=0)

    src = edge_index[1]
    dst = edge_index[0]
    w = edge_weight
    pad = EPAD_ - E_
    src_p = jnp.concatenate([src, jnp.zeros((pad,), jnp.int32)])
    dst_p = jnp.concatenate([dst, jnp.full((pad,), N_, jnp.int32)])
    w_p = jnp.concatenate([w, jnp.zeros((pad,), jnp.float32)])
    src2 = src_p.reshape(NB_, B_)
    dst2 = dst_p.reshape(NB_, B_)
    w2 = w_p.reshape(NB_, B_)

    _, out = _run(emb0_t, src2, dst2, w2)
    return (emb0, out)


# speedup vs baseline: 1.5403x; 1.2194x over previous
import jax
import jax.numpy as jnp
from jax import lax
from jax.experimental import pallas as pl
from jax.experimental.pallas import tpu as pltpu
from jax.experimental.pallas import tpu_sc as plsc

U_N_ = 60000
I_N_ = 40000
N_ = U_N_ + I_N_
D_ = 32
DH_ = 16
L_LAYERS_ = 3
E_ = 1600000
B_ = 128
NTILE_ = 16
G_ = 4

NB_ = E_ // B_
SP_BIG_ = 98
SP_SML_ = 95
NPAD_ = 100096
OROWS_ = NPAD_ // NTILE_
CB2_ = 2 * G_
MFULL_ = OROWS_ // B_
MREM_ = OROWS_ - MFULL_ * B_
MLAST_ = N_ - (NTILE_ - 1) * OROWS_ - MFULL_ * B_


def _body(emb0_t, ei3, w2,
          tbl, out2,
          acc, src_st, dst_st, w_st,
          ra0, ra1, ra2, ra3, rb0, rb1, rb2, rb3,
          gsa, gsb, ssa, ssb, stg):
    c = lax.axis_index("c")
    s = lax.axis_index("s")
    half_off = c * NPAD_

    grp_a = [ra0, ra1, ra2, ra3]
    grp_b = [rb0, rb1, rb2, rb3]
    r0 = s * OROWS_

    def zero_buf(buf):
        zv = jnp.zeros((16,), jnp.float32)

        def zrow(i, _):
            buf[i] = zv
            return 0

        lax.fori_loop(0, B_, zrow, 0)

    def zero_acc_slice():
        zero_buf(rb3)

        def zchunk(ch, _):
            pltpu.sync_copy(rb3, acc.at[pl.ds(r0 + ch * B_, B_)])
            return 0

        lax.fori_loop(0, MFULL_, zchunk, 0)
        pltpu.sync_copy(rb3.at[pl.ds(0, MREM_)],
                        acc.at[pl.ds(r0 + MFULL_ * B_, MREM_)])

    zero_acc_slice()

    plsc.subcore_barrier()

    def scale(buf, slot, jj):
        for m in range(B_ // 16):
            w16 = w_st[slot, jj, pl.ds(m * 16, 16)]
            for k in range(16):
                i = m * 16 + k
                buf[i] = buf[i] * w16[k]

    def edge_chunks(src_tab_full):
        src_tab = src_tab_full.at[pl.ds(pl.multiple_of(half_off, 8), NPAD_)]

        tb = SP_BIG_ * CB2_ * s - jnp.where(s == NTILE_ - 1, 3 * CB2_, 0)
        n_sp = jnp.where(s < NTILE_ - 2, SP_BIG_, SP_SML_)

        def g_start(slot, jg, bufs, sem):
            for b in range(G_):
                pltpu.async_copy(src_tab.at[src_st.at[slot, jg + b]],
                                 bufs[b], sem)

        def g_wait(bufs, sem):
            for b in range(G_):
                pltpu.make_async_copy(src_tab.at[src_st.at[0, 0]], bufs[b],
                                      sem).wait()

        def s_start(slot, jg, bufs, sem):
            for b in range(G_):
                pltpu.async_copy(bufs[b], acc.at[dst_st.at[slot, jg + b]],
                                 sem, add=True)

        def s_wait(bufs, sem):
            for b in range(G_):
                pltpu.make_async_copy(bufs[b], acc.at[dst_st.at[0, 0]],
                                      sem).wait()

        def scale4(slot, jg, bufs):
            for b in range(G_):
                scale(bufs[b], slot, jg + b)

        def stage_sync(q, slot):
            pltpu.sync_copy(ei3.at[1, pl.ds(tb + q * CB2_, CB2_)],
                            src_st.at[slot])
            pltpu.sync_copy(ei3.at[0, pl.ds(tb + q * CB2_, CB2_)],
                            dst_st.at[slot])
            pltpu.sync_copy(w2.at[pl.ds(tb + q * CB2_, CB2_)],
                            w_st.at[slot])

        def stage_async(q):
            slot = lax.rem(q, 3)
            pltpu.async_copy(ei3.at[1, pl.ds(tb + q * CB2_, CB2_)],
                             src_st.at[slot], stg)
            pltpu.async_copy(ei3.at[0, pl.ds(tb + q * CB2_, CB2_)],
                             dst_st.at[slot], stg)
            pltpu.async_copy(w2.at[pl.ds(tb + q * CB2_, CB2_)],
                             w_st.at[slot], stg)

        def stage_wait():
            pltpu.make_async_copy(ei3.at[1, pl.ds(0, CB2_)], src_st.at[0],
                                  stg).wait()
            pltpu.make_async_copy(ei3.at[0, pl.ds(0, CB2_)], dst_st.at[0],
                                  stg).wait()
            pltpu.make_async_copy(w2.at[pl.ds(0, CB2_)], w_st.at[0],
                                  stg).wait()

        stage_sync(0, 0)
        stage_async(1)
        g_start(0, 0, grp_a, gsa)

        def superpair(t, _):
            slot = lax.rem(t, 3)
            g_wait(grp_a, gsa)

            @pl.when(t > 0)
            def _():
                s_wait(grp_b, ssb)

            @pl.when(t + 2 <= n_sp - 1)
            def _():
                stage_async(t + 2)

            g_start(slot, G_, grp_b, gsb)
            scale4(slot, 0, grp_a)
            s_start(slot, 0, grp_a, ssa)
            g_wait(grp_b, gsb)
            s_wait(grp_a, ssa)

            @pl.when(t < n_sp - 1)
            def _():
                stage_wait()
                g_start(lax.rem(t + 1, 3), 0, grp_a, gsa)

            scale4(slot, G_, grp_b)
            s_start(slot, G_, grp_b, ssb)
            return 0

        lax.fori_loop(0, n_sp, superpair, 0)
        s_wait(grp_b, ssb)

        @pl.when(s == NTILE_ - 1)
        def _():
            t0 = NB_ - G_
            pltpu.sync_copy(ei3.at[1, pl.ds(t0, G_)],
                            src_st.at[0, pl.ds(0, G_)])
            pltpu.sync_copy(ei3.at[0, pl.ds(t0, G_)],
                            dst_st.at[0, pl.ds(0, G_)])
            pltpu.sync_copy(w2.at[pl.ds(t0, G_)],
                            w_st.at[0, pl.ds(0, G_)])
            g_start(0, 0, grp_a, gsa)
            g_wait(grp_a, gsa)
            scale4(0, 0, grp_a)
            s_start(0, 0, grp_a, ssa)
            s_wait(grp_a, ssa)

    def publish_and_rezero(layer):
        plsc.subcore_barrier()
        pltpu.sync_copy(acc.at[pl.ds(r0, OROWS_)],
                        tbl.at[layer, pl.ds(half_off + r0, OROWS_)])
        zero_acc_slice()
        plsc.subcore_barrier()

    edge_chunks(emb0_t)
    publish_and_rezero(0)

    def layer_step(lay, _):
        edge_chunks(tbl.at[lay])
        publish_and_rezero(lay + 1)
        return 0

    lax.fori_loop(0, L_LAYERS_ - 1, layer_step, 0)

    def mean_rows(nrows):
        def body(i, _):
            rb0[i] = (ra0[i] + ra1[i] + ra2[i] + ra3[i]) * 0.25
            return 0

        lax.fori_loop(0, nrows, body, 0)

    col0 = c * DH_

    def mean_chunk(ch, _):
        g0 = half_off + r0 + ch * B_
        gr = r0 + ch * B_
        pltpu.sync_copy(emb0_t.at[pl.ds(g0, B_)], ra0)
        pltpu.sync_copy(tbl.at[0, pl.ds(g0, B_)], ra1)
        pltpu.sync_copy(tbl.at[1, pl.ds(g0, B_)], ra2)
        pltpu.sync_copy(tbl.at[2, pl.ds(g0, B_)], ra3)
        mean_rows(B_)
        pltpu.sync_copy(rb0, out2.at[pl.ds(gr, B_), pl.ds(col0, DH_)])
        return 0

    lax.fori_loop(0, MFULL_, mean_chunk, 0)
    g0 = half_off + r0 + MFULL_ * B_
    gr = r0 + MFULL_ * B_
    pltpu.sync_copy(emb0_t.at[pl.ds(g0, MREM_)], ra0.at[pl.ds(0, MREM_)])
    pltpu.sync_copy(tbl.at[0, pl.ds(g0, MREM_)], ra1.at[pl.ds(0, MREM_)])
    pltpu.sync_copy(tbl.at[1, pl.ds(g0, MREM_)], ra2.at[pl.ds(0, MREM_)])
    pltpu.sync_copy(tbl.at[2, pl.ds(g0, MREM_)], ra3.at[pl.ds(0, MREM_)])
    mean_rows(MREM_)

    @pl.when(s < NTILE_ - 1)
    def _():
        pltpu.sync_copy(rb0.at[pl.ds(0, MREM_)],
                        out2.at[pl.ds(gr, MREM_), pl.ds(col0, DH_)])

    @pl.when(s == NTILE_ - 1)
    def _():
        pltpu.sync_copy(rb0.at[pl.ds(0, MLAST_)],
                        out2.at[pl.ds(gr, MLAST_), pl.ds(col0, DH_)])


@jax.jit
def _run(emb0_t, ei3, w2):
    mesh = plsc.VectorSubcoreMesh(core_axis_name="c", subcore_axis_name="s")
    f32 = jnp.float32
    out_types = (
        jax.ShapeDtypeStruct((3, 2 * NPAD_, DH_), f32),
        jax.ShapeDtypeStruct((N_, D_), f32),
    )
    scratch = [
        pltpu.VMEM_SHARED((NPAD_, DH_), f32),
        pltpu.VMEM((3, CB2_, B_), jnp.int32),
        pltpu.VMEM((3, CB2_, B_), jnp.int32),
        pltpu.VMEM((3, CB2_, B_), f32),
    ]
    scratch += [pltpu.VMEM((B_, DH_), f32) for _ in range(8)]
    scratch += [pltpu.SemaphoreType.DMA] * 5
    kfn = pl.kernel(
        _body,
        out_type=out_types,
        scratch_types=scratch,
        mesh=mesh,
        compiler_params=pltpu.CompilerParams(use_tc_tiling_on_sc=False),
    )
    return kfn(emb0_t, ei3, w2)


def kernel(edge_index, edge_weight, W_user, W_item):
    emb0 = jnp.concatenate([W_user, W_item], axis=0)
    rpad = jnp.zeros((NPAD_ - N_, DH_), jnp.float32)
    emb0_t = jnp.concatenate(
        [emb0[:, :DH_], rpad, emb0[:, DH_:], rpad], axis=0)

    ei3 = edge_index.reshape(2, NB_, B_)
    w2 = edge_weight.reshape(NB_, B_)

    _, out = _run(emb0_t, ei3, w2)
    return (emb0, out)
